# baseline (speedup 1.0000x reference)
import functools

import jax
import jax.numpy as jnp
from jax import lax
from jax.experimental import pallas as pl
from jax.experimental.pallas import tpu as pltpu
from jax.experimental.pallas import tpu_sc as plsc

VOCAB = 256
DIM = 768
GATHER_ROWS = 80
N_HALVES = 2


def _make_kernel(batch: int, seq: int):
  info = plsc.get_sparse_core_info()
  nc, ns = info.num_cores, info.num_subcores
  nw = nc * ns
  per_w = batch // nw
  half = per_w // N_HALVES
  n_pairs = half // 2
  assert batch % (2 * N_HALVES * nw) == 0
  aligned = (seq // 8) * 8

  mesh = plsc.VectorSubcoreMesh(core_axis_name="c", subcore_axis_name="s")

  @functools.partial(
      pl.kernel,
      out_type=jax.ShapeDtypeStruct((batch, seq, DIM), jnp.float32),
      mesh=mesh,
      scratch_types=[
          pltpu.VMEM((half * GATHER_ROWS,), jnp.int32),
          pltpu.VMEM((GATHER_ROWS, DIM), jnp.float32),
          pltpu.VMEM((GATHER_ROWS, DIM), jnp.float32),
          pltpu.SemaphoreType.DMA,
          pltpu.SemaphoreType.DMA,
      ],
      compiler_params=pltpu.CompilerParams(use_tc_tiling_on_sc=True),
  )
  def gather_kernel(ids_hbm, table_hbm, out_hbm,
                    idx_blk, rows0, rows1, sem0, sem1):
    c = lax.axis_index("c")
    s = lax.axis_index("s")
    wid = s * nc + c
    r0 = wid * per_w

    def glist(j):
      return idx_blk.at[pl.ds(j * GATHER_ROWS, GATHER_ROWS)]

    def write_panel(rows_v, b):
      pltpu.sync_copy(rows_v.at[pl.ds(0, aligned)],
                      out_hbm.at[b, pl.ds(0, aligned)])
      for t in range(aligned, seq):
        pltpu.sync_copy(rows_v.at[pl.ds(t, 1)], out_hbm.at[b, pl.ds(t, 1)])

    for h in range(N_HALVES):
      b0 = r0 + h * half
      pltpu.sync_copy(ids_hbm.at[pl.ds(b0 * GATHER_ROWS, half * GATHER_ROWS)],
                      idx_blk)
      pltpu.async_copy(table_hbm.at[glist(0)], rows0, sem0)

      def body(i, carry):
        j = 2 * i
        b = b0 + j
        pltpu.async_copy(table_hbm.at[glist(j + 1)], rows1, sem1)
        pltpu.make_async_copy(table_hbm.at[glist(j)], rows0, sem0).wait()
        write_panel(rows0, b)

        @pl.when(i < n_pairs - 1)
        def _():
          pltpu.async_copy(table_hbm.at[glist(j + 2)], rows0, sem0)

        pltpu.make_async_copy(table_hbm.at[glist(j + 1)], rows1, sem1).wait()
        write_panel(rows1, b + 1)
        return carry

      lax.fori_loop(0, n_pairs, body, 0)

  return gather_kernel


def kernel(token_ids, table):
  b, t = token_ids.shape
  ids_pad = jnp.pad(token_ids.astype(jnp.int32),
                    ((0, 0), (0, GATHER_ROWS - t)))
  flat = ids_pad.reshape(b * GATHER_ROWS)
  return _make_kernel(b, t)(flat, table)
